# baseline (speedup 1.0000x reference)
import jax
import jax.numpy as jnp
from jax.experimental import pallas as pl
from jax.experimental.pallas import tpu as pltpu

D_MODEL = 1024
D_FF = 4096
N_EXPERTS = 8
SEQ = 2048
TOP_K = 2

TM_S = 512
TF = 1024
J = D_FF // TF
NT = (SEQ * TOP_K) // TM_S + N_EXPERTS
PAD = NT * TM_S


def _router_kernel(xb_ref, rw_ref, coef_ref):
    logits = jnp.dot(xb_ref[...], rw_ref[...],
                     preferred_element_type=jnp.float32)
    mx = jnp.max(logits, axis=-1, keepdims=True)
    ex = jnp.exp(logits - mx)
    p = ex / jnp.sum(ex, axis=-1, keepdims=True)
    lane = jax.lax.broadcasted_iota(jnp.int32, p.shape, 1)
    m1 = jnp.max(p, axis=-1, keepdims=True)
    i1 = jnp.min(jnp.where(p == m1, lane, N_EXPERTS), axis=-1, keepdims=True)
    mask1 = lane == i1
    pm = jnp.where(mask1, -1.0, p)
    m2 = jnp.max(pm, axis=-1, keepdims=True)
    i2 = jnp.min(jnp.where(pm == m2, lane, N_EXPERTS), axis=-1, keepdims=True)
    mask2 = lane == i2
    coef_ref[...] = jnp.where(mask1 | mask2, p, 0.0) / (m1 + m2)


def _moe_sparse_kernel(te_ref, tv_ref, x_ref, rt_ref, rwt_ref,
                       w1_ref, w2_ref, w3_ref, out_ref, gt_ref, xg_ref,
                       acc_ref):
    i = pl.program_id(0)
    j = pl.program_id(1)

    @pl.when((i == 0) & (j == 0))
    def _init():
        out_ref[...] = jnp.zeros_like(out_ref)

    @pl.when(tv_ref[i] == 1)
    def _work():

        @pl.when(j == 0)
        def _gather():
            rt = rt_ref[0]
            t_iota = jax.lax.broadcasted_iota(jnp.int32, (SEQ, TM_S), 0)
            gt = (t_iota == rt).astype(jnp.bfloat16)
            gt_ref[...] = gt
            xg_ref[...] = jax.lax.dot_general(
                gt, x_ref[...], (((0,), (0,)), ((), ())),
                preferred_element_type=jnp.float32).astype(jnp.bfloat16)
            acc_ref[...] = jnp.zeros_like(acc_ref)

        xg = xg_ref[...]
        gate = jnp.dot(xg, w1_ref[0], preferred_element_type=jnp.float32)
        val = jnp.dot(xg, w2_ref[0], preferred_element_type=jnp.float32)
        h = (gate * jax.nn.sigmoid(gate) * val).astype(jnp.bfloat16)
        acc_ref[...] += jnp.dot(h, w3_ref[0], preferred_element_type=jnp.float32)

        @pl.when(j == J - 1)
        def _scatter():
            w_row = rwt_ref[0].astype(jnp.bfloat16)
            gw = gt_ref[...] * w_row
            y = acc_ref[...].astype(jnp.bfloat16)
            out_ref[...] += jnp.dot(gw, y, preferred_element_type=jnp.float32)


def kernel(x, router_weight, w1, w2, w3):
    batch, seq, d = x.shape
    xb = x.reshape(seq, d).astype(jnp.bfloat16)
    rwb = router_weight.astype(jnp.bfloat16)
    w1b = w1.astype(jnp.bfloat16)
    w2b = w2.astype(jnp.bfloat16)
    w3b = w3.astype(jnp.bfloat16)

    coef = pl.pallas_call(
        _router_kernel,
        in_specs=[
            pl.BlockSpec((seq, d), lambda: (0, 0)),
            pl.BlockSpec((d, N_EXPERTS), lambda: (0, 0)),
        ],
        out_specs=pl.BlockSpec((seq, N_EXPERTS), lambda: (0, 0)),
        out_shape=jax.ShapeDtypeStruct((seq, N_EXPERTS), jnp.float32),
    )(xb, rwb)

    mask = coef > 0.0
    mi = mask.astype(jnp.int32)
    incl = jnp.cumsum(mi, axis=0)
    rank = incl - mi
    counts = incl[-1]
    aligned = ((counts + TM_S - 1) // TM_S) * TM_S
    acum = jnp.cumsum(aligned)
    astart = acum - aligned
    pos = jnp.where(mask, astart[None, :] + rank, PAD)
    pos_flat = pos.reshape(-1)
    tok = jax.lax.broadcasted_iota(jnp.int32, (seq, N_EXPERTS), 0).reshape(-1)
    row_token = jnp.zeros((PAD,), jnp.int32).at[pos_flat].set(
        tok, mode="drop")
    row_weight = jnp.zeros((PAD,), jnp.float32).at[pos_flat].set(
        coef.reshape(-1), mode="drop")
    tile_start = jnp.arange(NT, dtype=jnp.int32) * TM_S
    tile_expert = jnp.minimum(
        jnp.searchsorted(acum, tile_start, side="right").astype(jnp.int32),
        N_EXPERTS - 1)
    tile_valid = (tile_start < acum[-1]).astype(jnp.int32)

    out = pl.pallas_call(
        _moe_sparse_kernel,
        grid_spec=pltpu.PrefetchScalarGridSpec(
            num_scalar_prefetch=2,
            grid=(NT, J),
            in_specs=[
                pl.BlockSpec((seq, d), lambda i, j, te, tv: (0, 0)),
                pl.BlockSpec((1, 1, TM_S), lambda i, j, te, tv: (i, 0, 0)),
                pl.BlockSpec((1, 1, TM_S), lambda i, j, te, tv: (i, 0, 0)),
                pl.BlockSpec((1, D_MODEL, TF),
                             lambda i, j, te, tv:
                             (te[i], 0, jnp.where(tv[i] == 1, j, J - 1))),
                pl.BlockSpec((1, D_MODEL, TF),
                             lambda i, j, te, tv:
                             (te[i], 0, jnp.where(tv[i] == 1, j, J - 1))),
                pl.BlockSpec((1, TF, D_MODEL),
                             lambda i, j, te, tv:
                             (te[i], jnp.where(tv[i] == 1, j, J - 1), 0)),
            ],
            out_specs=pl.BlockSpec((seq, d), lambda i, j, te, tv: (0, 0)),
            scratch_shapes=[
                pltpu.VMEM((SEQ, TM_S), jnp.bfloat16),
                pltpu.VMEM((TM_S, D_MODEL), jnp.bfloat16),
                pltpu.VMEM((TM_S, D_MODEL), jnp.float32),
            ],
        ),
        out_shape=jax.ShapeDtypeStruct((seq, d), jnp.float32),
        compiler_params=pltpu.CompilerParams(
            dimension_semantics=("arbitrary", "arbitrary"),
        ),
    )(tile_expert, tile_valid, xb,
      row_token.reshape(NT, 1, TM_S), row_weight.reshape(NT, 1, TM_S),
      w1b, w2b, w3b)
    return out.reshape(batch, seq, d)


# speedup vs baseline: 1.1830x; 1.1830x over previous
import jax
import jax.numpy as jnp
from jax.experimental import pallas as pl
from jax.experimental.pallas import tpu as pltpu

D_MODEL = 1024
D_FF = 4096
N_EXPERTS = 8
SEQ = 2048
TOP_K = 2

TM_S = 512
TF = 1024
J = D_FF // TF
NT = (SEQ * TOP_K) // TM_S + N_EXPERTS
PAD = NT * TM_S


def _router_kernel(xb_ref, rw_ref, posw_ref, acum_ref):
    logits = jnp.dot(xb_ref[...], rw_ref[...],
                     preferred_element_type=jnp.float32)
    mx = jnp.max(logits, axis=-1, keepdims=True)
    ex = jnp.exp(logits - mx)
    p = ex / jnp.sum(ex, axis=-1, keepdims=True)
    lane = jax.lax.broadcasted_iota(jnp.int32, p.shape, 1)
    m1 = jnp.max(p, axis=-1, keepdims=True)
    i1 = jnp.min(jnp.where(p == m1, lane, N_EXPERTS), axis=-1, keepdims=True)
    mask1 = lane == i1
    pm = jnp.where(mask1, -1.0, p)
    m2 = jnp.max(pm, axis=-1, keepdims=True)
    i2 = jnp.min(jnp.where(pm == m2, lane, N_EXPERTS), axis=-1, keepdims=True)
    mask2 = lane == i2
    denom = m1 + m2
    w1n = m1 / denom
    w2n = m2 / denom

    mu = (mask1 | mask2).astype(jnp.bfloat16)
    r_io = jax.lax.broadcasted_iota(jnp.int32, (SEQ, SEQ), 0)
    c_io = jax.lax.broadcasted_iota(jnp.int32, (SEQ, SEQ), 1)
    tril = (r_io > c_io).astype(jnp.bfloat16)
    rank = jax.lax.dot_general(tril, mu, (((1,), (0,)), ((), ())),
                               preferred_element_type=jnp.float32)
    counts = rank[SEQ - 1:SEQ, :] + mu[SEQ - 1:SEQ, :].astype(jnp.float32)

    ntiles = jnp.ceil(counts / TM_S)
    r8 = jax.lax.broadcasted_iota(jnp.int32, (N_EXPERTS, N_EXPERTS), 0)
    c8 = jax.lax.broadcasted_iota(jnp.int32, (N_EXPERTS, N_EXPERTS), 1)
    uppi = (r8 <= c8).astype(jnp.bfloat16)
    acum = jax.lax.dot_general(ntiles.astype(jnp.bfloat16), uppi,
                               (((1,), (0,)), ((), ())),
                               preferred_element_type=jnp.float32) * TM_S
    astart = acum - ntiles * TM_S

    pos = astart + rank
    pos1 = jnp.sum(jnp.where(mask1, pos, 0.0), axis=-1, keepdims=True)
    pos2 = jnp.sum(jnp.where(mask2, pos, 0.0), axis=-1, keepdims=True)
    posw_ref[...] = (jnp.where(lane == 0, pos1, 0.0)
                     + jnp.where(lane == 1, pos2, 0.0)
                     + jnp.where(lane == 2, w1n, 0.0)
                     + jnp.where(lane == 3, w2n, 0.0))
    acum_ref[...] = acum


def _moe_sparse_kernel(te_ref, tv_ref, x_ref, rt_ref, rwt_ref,
                       w1_ref, w2_ref, w3_ref, out_ref, gt_ref, xg_ref,
                       acc_ref):
    i = pl.program_id(0)
    j = pl.program_id(1)

    @pl.when((i == 0) & (j == 0))
    def _init():
        out_ref[...] = jnp.zeros_like(out_ref)

    @pl.when(tv_ref[i] == 1)
    def _work():

        @pl.when(j == 0)
        def _gather():
            rt = rt_ref[0]
            t_iota = jax.lax.broadcasted_iota(jnp.int32, (SEQ, TM_S), 0)
            gt = (t_iota == rt).astype(jnp.bfloat16)
            gt_ref[...] = gt
            xg_ref[...] = jax.lax.dot_general(
                gt, x_ref[...], (((0,), (0,)), ((), ())),
                preferred_element_type=jnp.float32).astype(jnp.bfloat16)
            acc_ref[...] = jnp.zeros_like(acc_ref)

        xg = xg_ref[...]
        gate = jnp.dot(xg, w1_ref[0], preferred_element_type=jnp.float32)
        val = jnp.dot(xg, w2_ref[0], preferred_element_type=jnp.float32)
        h = (gate * jax.nn.sigmoid(gate) * val).astype(jnp.bfloat16)
        acc_ref[...] += jnp.dot(h, w3_ref[0], preferred_element_type=jnp.float32)

        @pl.when(j == J - 1)
        def _scatter():
            w_row = rwt_ref[0].astype(jnp.bfloat16)
            gw = gt_ref[...] * w_row
            y = acc_ref[...].astype(jnp.bfloat16)
            out_ref[...] += jnp.dot(gw, y, preferred_element_type=jnp.float32)


def kernel(x, router_weight, w1, w2, w3):
    batch, seq, d = x.shape
    xb = x.reshape(seq, d).astype(jnp.bfloat16)
    rwb = router_weight.astype(jnp.bfloat16)
    w1b = w1.astype(jnp.bfloat16)
    w2b = w2.astype(jnp.bfloat16)
    w3b = w3.astype(jnp.bfloat16)

    posw, acum_f = pl.pallas_call(
        _router_kernel,
        in_specs=[
            pl.BlockSpec((seq, d), lambda: (0, 0)),
            pl.BlockSpec((d, N_EXPERTS), lambda: (0, 0)),
        ],
        out_specs=[
            pl.BlockSpec((seq, N_EXPERTS), lambda: (0, 0)),
            pl.BlockSpec((1, N_EXPERTS), lambda: (0, 0)),
        ],
        out_shape=[
            jax.ShapeDtypeStruct((seq, N_EXPERTS), jnp.float32),
            jax.ShapeDtypeStruct((1, N_EXPERTS), jnp.float32),
        ],
    )(xb, rwb)

    pos_flat = posw[:, :TOP_K].astype(jnp.int32).reshape(-1)
    tok = (jnp.arange(seq * TOP_K, dtype=jnp.int32) // TOP_K)
    row_token = jnp.zeros((PAD,), jnp.int32).at[pos_flat].set(tok)
    row_weight = jnp.zeros((PAD,), jnp.float32).at[pos_flat].set(
        posw[:, TOP_K:2 * TOP_K].reshape(-1))
    acum = acum_f[0].astype(jnp.int32)
    tile_start = jnp.arange(NT, dtype=jnp.int32) * TM_S
    tile_expert = jnp.minimum(
        jnp.sum((acum[None, :] <= tile_start[:, None]).astype(jnp.int32),
                axis=1),
        N_EXPERTS - 1)
    tile_valid = (tile_start < acum[-1]).astype(jnp.int32)

    out = pl.pallas_call(
        _moe_sparse_kernel,
        grid_spec=pltpu.PrefetchScalarGridSpec(
            num_scalar_prefetch=2,
            grid=(NT, J),
            in_specs=[
                pl.BlockSpec((seq, d), lambda i, j, te, tv: (0, 0)),
                pl.BlockSpec((1, 1, TM_S), lambda i, j, te, tv: (i, 0, 0)),
                pl.BlockSpec((1, 1, TM_S), lambda i, j, te, tv: (i, 0, 0)),
                pl.BlockSpec((1, D_MODEL, TF),
                             lambda i, j, te, tv:
                             (te[i], 0, jnp.where(tv[i] == 1, j, J - 1))),
                pl.BlockSpec((1, D_MODEL, TF),
                             lambda i, j, te, tv:
                             (te[i], 0, jnp.where(tv[i] == 1, j, J - 1))),
                pl.BlockSpec((1, TF, D_MODEL),
                             lambda i, j, te, tv:
                             (te[i], jnp.where(tv[i] == 1, j, J - 1), 0)),
            ],
            out_specs=pl.BlockSpec((seq, d), lambda i, j, te, tv: (0, 0)),
            scratch_shapes=[
                pltpu.VMEM((SEQ, TM_S), jnp.bfloat16),
                pltpu.VMEM((TM_S, D_MODEL), jnp.bfloat16),
                pltpu.VMEM((TM_S, D_MODEL), jnp.float32),
            ],
        ),
        out_shape=jax.ShapeDtypeStruct((seq, d), jnp.float32),
        compiler_params=pltpu.CompilerParams(
            dimension_semantics=("arbitrary", "arbitrary"),
        ),
    )(tile_expert, tile_valid, xb,
      row_token.reshape(NT, 1, TM_S), row_weight.reshape(NT, 1, TM_S),
      w1b, w2b, w3b)
    return out.reshape(batch, seq, d)
